# baseline (speedup 1.0000x reference)
import jax
import jax.numpy as jnp
from jax import lax
from jax.experimental import pallas as pl
from jax.experimental.pallas import tpu as pltpu
from jax.experimental.pallas import tpu_sc as plsc

BATCH = 16384
F = 32
NROWS = 1000000
NC = 2
NS = 16
NW = NC * NS
BPW = BATCH // NW
L = 16
CHUNKS = BPW // L

TCOLS = NROWS // 128
NTILES = 4 * TCOLS
NBLK = NTILES // 8
XROWS = (NBLK + 1) * 64
TAIL = TCOLS * 128
BLK_PER_W = (NBLK + NW - 1) // NW


def _detile_body(ift_hbm, xi_hbm,
                 bi0_v, bi1_v,
                 sin_i0, sin_i1, soi0, soi1):
    wid = lax.axis_index("s") * NC + lax.axis_index("c")
    sets = ((bi0_v, sin_i0, soi0), (bi1_v, sin_i1, soi1))

    def fire_in(k, bi, si):
        for j in range(8):
            g = j % 4
            coff = pl.multiple_of((2 * k + j // 4) * 128, 128)
            pltpu.async_copy(ift_hbm.at[pl.ds(8 * g, 8), pl.ds(coff, 128)],
                             bi.at[pl.ds(8 * j, 8), :], si)

    def flush(k, bi, si, so_i):
        pltpu.make_async_copy(xi_hbm.at[pl.ds(0, 64), :], bi, si).wait()
        rowoff = pl.multiple_of(64 * k, 8)
        pltpu.async_copy(bi, xi_hbm.at[pl.ds(rowoff, 64), :], so_i)

    def drain_out(bi, so_i):
        pltpu.make_async_copy(xi_hbm.at[pl.ds(0, 64), :], bi, so_i).wait()

    def pair(kk2, _):
        ks = (wid + NW * 2 * kk2, wid + NW * (2 * kk2 + 1))
        prev = (ks[0] - 2 * NW, ks[1] - 2 * NW)
        for d in range(2):
            @pl.when((prev[d] >= 0) & (prev[d] < NBLK))
            def _(d=d):
                bi, _, so_i = sets[d]
                drain_out(bi, so_i)
        for d in range(2):
            @pl.when(ks[d] < NBLK)
            def _(d=d):
                bi, si, _ = sets[d]
                fire_in(ks[d], bi, si)
        for d in range(2):
            @pl.when(ks[d] < NBLK)
            def _(d=d):
                flush(ks[d], *sets[d])
        return 0

    npair = (BLK_PER_W + 1) // 2
    lax.fori_loop(0, npair, pair, 0)
    for d in range(2):
        last = wid + NW * (2 * (npair - 1) + d)

        @pl.when(last < NBLK)
        def _(d=d, last=last):
            bi, _, so_i = sets[d]
            drain_out(bi, so_i)


def _tc_detile_body(in_ref, out_ref):
    for j in range(8):
        out_ref[pl.ds(8 * j, 8), :] = (
            in_ref[pl.ds(8 * (j % 4), 8), pl.ds(128 * (j // 4), 128)])


def _gather_body(xu_hbm, xi_hbm, ub_hbm, ib_hbm, user_hbm, item_hbm,
                 auxu_hbm, auxi_hbm, out_hbm,
                 uidx_v, iidx_v, idxu_v, idxi_v, upl_v, ipl_v,
                 ub_v, ib_v, auxu_v, auxi_v, out_v,
                 sem_u, sem_i, sem_b, sem_s):
    wid = lax.axis_index("s") * NC + lax.axis_index("c")
    base = wid * BPW

    pltpu.async_copy(user_hbm.at[pl.ds(base, BPW)], uidx_v, sem_s).wait()
    pltpu.async_copy(item_hbm.at[pl.ds(base, BPW)], iidx_v, sem_s).wait()
    pltpu.async_copy(auxu_hbm, auxu_v, sem_s).wait()
    pltpu.async_copy(auxi_hbm, auxi_v, sem_s).wait()

    cb0 = pltpu.async_copy(ub_hbm.at[uidx_v], ub_v, sem_b)
    cb1 = pltpu.async_copy(ib_hbm.at[iidx_v], ib_v, sem_b)

    offs = [(f >> 3) * 1024 + (f & 7) * 128 for f in range(F)]

    def flat_base(r):
        rc = jnp.minimum(r, TAIL - 1)
        return (rc >> 8) * 8192 + ((rc >> 7) & 1) * 4096 + (rc & 127)

    def compute_idx(c, _):
        s = pl.ds(c * L, L)
        bu = flat_base(uidx_v[s])
        bi = flat_base(iidx_v[s])
        for f in range(F):
            idxu_v[f, s] = bu + offs[f]
            idxi_v[f, s] = bi + offs[f]
        return 0

    lax.fori_loop(0, CHUNKS, compute_idx, 0)

    plane_copies = []
    for f in range(F):
        plane_copies.append(
            pltpu.async_copy(xu_hbm.at[idxu_v.at[f]], upl_v.at[f], sem_u))
        plane_copies.append(
            pltpu.async_copy(xi_hbm.at[idxi_v.at[f]], ipl_v.at[f], sem_i))
    for c in plane_copies:
        c.wait()
    cb0.wait()
    cb1.wait()

    iota = lax.iota(jnp.int32, L)

    def chunk(c, _):
        s = pl.ds(c * L, L)
        ru = uidx_v[s]
        ri = iidx_v[s]
        acc = ub_v[s] + ib_v[s]
        for f in range(F):
            acc = acc + upl_v[f, s] * ipl_v[f, s]
        out_v[s] = acc

        tu = ru >= TAIL
        ti = ri >= TAIL
        any_tail = jnp.max(jnp.where(tu | ti, 1, 0))

        @pl.when(any_tail > 0)
        def _fixup():
            du = jnp.clip(ru - TAIL, 0, 63)
            di = jnp.clip(ri - TAIL, 0, 63)
            acc2 = ub_v[s] + ib_v[s]
            for f in range(F):
                au = plsc.load_gather(auxu_v, [f * 64 + du])
                ai = plsc.load_gather(auxi_v, [f * 64 + di])
                u = jnp.where(tu, au, upl_v[f, s])
                it = jnp.where(ti, ai, ipl_v[f, s])
                acc2 = acc2 + u * it
            out_v[s] = acc2

        return 0

    lax.fori_loop(0, CHUNKS, chunk, 0)

    pltpu.sync_copy(out_v, out_hbm.at[pl.ds(base, BPW)])


@jax.jit
def _mf(user, item, user_factors, item_factors, user_biases, item_biases):
    mesh = plsc.VectorSubcoreMesh(core_axis_name="c", subcore_axis_name="s")

    sc_detile = pl.kernel(
        _detile_body,
        out_type=jax.ShapeDtypeStruct((XROWS, 128), jnp.float32),
        mesh=mesh,
        compiler_params=pltpu.CompilerParams(use_tc_tiling_on_sc=True),
        scratch_types=[
            pltpu.VMEM((64, 128), jnp.float32),
            pltpu.VMEM((64, 128), jnp.float32),
            pltpu.SemaphoreType.DMA,
            pltpu.SemaphoreType.DMA,
            pltpu.SemaphoreType.DMA,
            pltpu.SemaphoreType.DMA,
        ],
    )
    xi = sc_detile(item_factors.T)

    tc_detile = pl.pallas_call(
        _tc_detile_body,
        grid=(NBLK,),
        in_specs=[pl.BlockSpec((F, 256), lambda i: (0, i))],
        out_specs=pl.BlockSpec((64, 128), lambda i: (i, 0)),
        out_shape=jax.ShapeDtypeStruct((NBLK * 64, 128), jnp.float32),
    )
    xu = tc_detile(user_factors.T)

    gather = pl.kernel(
        _gather_body,
        out_type=jax.ShapeDtypeStruct((BATCH,), jnp.float32),
        mesh=mesh,
        compiler_params=pltpu.CompilerParams(
            needs_layout_passes=False, use_tc_tiling_on_sc=False),
        scratch_types=[
            pltpu.VMEM((BPW,), jnp.int32),
            pltpu.VMEM((BPW,), jnp.int32),
            pltpu.VMEM((F, BPW), jnp.int32),
            pltpu.VMEM((F, BPW), jnp.int32),
            pltpu.VMEM((F, BPW), jnp.float32),
            pltpu.VMEM((F, BPW), jnp.float32),
            pltpu.VMEM((BPW,), jnp.float32),
            pltpu.VMEM((BPW,), jnp.float32),
            pltpu.VMEM((64 * F,), jnp.float32),
            pltpu.VMEM((64 * F,), jnp.float32),
            pltpu.VMEM((BPW,), jnp.float32),
            pltpu.SemaphoreType.DMA,
            pltpu.SemaphoreType.DMA,
            pltpu.SemaphoreType.DMA,
            pltpu.SemaphoreType.DMA,
        ],
    )
    auxu = user_factors[TAIL:, :].T.reshape(-1)
    auxi = item_factors[TAIL:, :].T.reshape(-1)
    return gather(xu.reshape(-1), xi.reshape(-1),
                  user_biases.reshape(-1), item_biases.reshape(-1),
                  user, item, auxu, auxi)


def kernel(user, item, user_factors, item_factors, user_biases, item_biases):
    return _mf(user, item, user_factors, item_factors, user_biases, item_biases)


# speedup vs baseline: 7.4976x; 7.4976x over previous
import jax
import jax.numpy as jnp
from jax import lax
from jax.experimental import pallas as pl
from jax.experimental.pallas import tpu as pltpu
from jax.experimental.pallas import tpu_sc as plsc

BATCH = 16384
F = 32
NROWS = 1000000
NC = 2
NS = 16
NW = NC * NS
BPW = BATCH // NW
L = 16
CHUNKS = BPW // L

TCOLS = NROWS // 128
NTILES = 4 * TCOLS
NBLK = NTILES // 8
XROWS = (NBLK + 1) * 64
TAIL = TCOLS * 128
BLK_PER_W = (NBLK + NW - 1) // NW


def _detile_body(uft_hbm, ift_hbm, xu_hbm, xi_hbm,
                 bu0_v, bi0_v, bu1_v, bi1_v,
                 sin_u0, sin_i0, sin_u1, sin_i1, sou0, soi0, sou1, soi1):
    wid = lax.axis_index("s") * NC + lax.axis_index("c")
    sets = ((bu0_v, bi0_v, sin_u0, sin_i0, sou0, soi0),
            (bu1_v, bi1_v, sin_u1, sin_i1, sou1, soi1))

    def fire_in(k, bu, bi, su, si):
        for j in range(8):
            g = j % 4
            coff = pl.multiple_of((2 * k + j // 4) * 128, 128)
            pltpu.async_copy(uft_hbm.at[pl.ds(8 * g, 8), pl.ds(coff, 128)],
                             bu.at[pl.ds(8 * j, 8), :], su)
            pltpu.async_copy(ift_hbm.at[pl.ds(8 * g, 8), pl.ds(coff, 128)],
                             bi.at[pl.ds(8 * j, 8), :], si)

    def flush(k, bu, bi, su, si, so_u, so_i):
        pltpu.make_async_copy(xu_hbm.at[pl.ds(0, 64), :], bu, su).wait()
        pltpu.make_async_copy(xi_hbm.at[pl.ds(0, 64), :], bi, si).wait()
        rowoff = pl.multiple_of(64 * k, 8)
        pltpu.async_copy(bu, xu_hbm.at[pl.ds(rowoff, 64), :], so_u)
        pltpu.async_copy(bi, xi_hbm.at[pl.ds(rowoff, 64), :], so_i)

    def drain_out(bu, bi, so_u, so_i):
        pltpu.make_async_copy(xu_hbm.at[pl.ds(0, 64), :], bu, so_u).wait()
        pltpu.make_async_copy(xi_hbm.at[pl.ds(0, 64), :], bi, so_i).wait()

    def pair(kk2, _):
        ks = (wid + NW * 2 * kk2, wid + NW * (2 * kk2 + 1))
        for d in range(2):
            @pl.when(ks[d] < NBLK)
            def _(d=d):
                bu, bi, su, si, _, _ = sets[d]
                fire_in(ks[d], bu, bi, su, si)
        for d in range(2):
            @pl.when(ks[d] < NBLK)
            def _(d=d):
                flush(ks[d], *sets[d])
        for d in range(2):
            @pl.when(ks[d] < NBLK)
            def _(d=d):
                bu, bi, _, _, so_u, so_i = sets[d]
                drain_out(bu, bi, so_u, so_i)
        return 0

    lax.fori_loop(0, (BLK_PER_W + 1) // 2, pair, 0)


def _gather_body(xu_hbm, xi_hbm, ub_hbm, ib_hbm, user_hbm, item_hbm,
                 auxu_hbm, auxi_hbm, out_hbm,
                 uidx_v, iidx_v, idxu_v, idxi_v, upl_v, ipl_v,
                 ub_v, ib_v, auxu_v, auxi_v, out_v,
                 sem_u, sem_i, sem_b, sem_s):
    wid = lax.axis_index("s") * NC + lax.axis_index("c")
    base = wid * BPW

    pltpu.async_copy(user_hbm.at[pl.ds(base, BPW)], uidx_v, sem_s).wait()
    pltpu.async_copy(item_hbm.at[pl.ds(base, BPW)], iidx_v, sem_s).wait()
    pltpu.async_copy(auxu_hbm, auxu_v, sem_s).wait()
    pltpu.async_copy(auxi_hbm, auxi_v, sem_s).wait()

    cb0 = pltpu.async_copy(ub_hbm.at[uidx_v], ub_v, sem_b)
    cb1 = pltpu.async_copy(ib_hbm.at[iidx_v], ib_v, sem_b)

    offs = [(f >> 3) * 1024 + (f & 7) * 128 for f in range(F)]

    def flat_base(r):
        rc = jnp.minimum(r, TAIL - 1)
        return (rc >> 8) * 8192 + ((rc >> 7) & 1) * 4096 + (rc & 127)

    def compute_idx(c, _):
        s = pl.ds(c * L, L)
        bu = flat_base(uidx_v[s])
        bi = flat_base(iidx_v[s])
        for f in range(F):
            idxu_v[f, s] = bu + offs[f]
            idxi_v[f, s] = bi + offs[f]
        return 0

    lax.fori_loop(0, CHUNKS, compute_idx, 0)

    plane_copies = []
    for f in range(F):
        plane_copies.append(
            pltpu.async_copy(xu_hbm.at[idxu_v.at[f]], upl_v.at[f], sem_u))
        plane_copies.append(
            pltpu.async_copy(xi_hbm.at[idxi_v.at[f]], ipl_v.at[f], sem_i))
    for c in plane_copies:
        c.wait()
    cb0.wait()
    cb1.wait()

    iota = lax.iota(jnp.int32, L)

    def chunk(c, _):
        s = pl.ds(c * L, L)
        ru = uidx_v[s]
        ri = iidx_v[s]
        acc = ub_v[s] + ib_v[s]
        for f in range(F):
            acc = acc + upl_v[f, s] * ipl_v[f, s]
        out_v[s] = acc

        tu = ru >= TAIL
        ti = ri >= TAIL
        any_tail = jnp.max(jnp.where(tu | ti, 1, 0))

        @pl.when(any_tail > 0)
        def _fixup():
            du = jnp.clip(ru - TAIL, 0, 63)
            di = jnp.clip(ri - TAIL, 0, 63)
            acc2 = ub_v[s] + ib_v[s]
            for f in range(F):
                au = plsc.load_gather(auxu_v, [f * 64 + du])
                ai = plsc.load_gather(auxi_v, [f * 64 + di])
                u = jnp.where(tu, au, upl_v[f, s])
                it = jnp.where(ti, ai, ipl_v[f, s])
                acc2 = acc2 + u * it
            out_v[s] = acc2

        return 0

    lax.fori_loop(0, CHUNKS, chunk, 0)

    pltpu.sync_copy(out_v, out_hbm.at[pl.ds(base, BPW)])


@jax.jit
def _mf(user, item, user_factors, item_factors, user_biases, item_biases):
    mesh = plsc.VectorSubcoreMesh(core_axis_name="c", subcore_axis_name="s")

    detile = pl.kernel(
        _detile_body,
        out_type=(jax.ShapeDtypeStruct((XROWS, 128), jnp.float32),
                  jax.ShapeDtypeStruct((XROWS, 128), jnp.float32)),
        mesh=mesh,
        compiler_params=pltpu.CompilerParams(use_tc_tiling_on_sc=True),
        scratch_types=[
            pltpu.VMEM((64, 128), jnp.float32),
            pltpu.VMEM((64, 128), jnp.float32),
            pltpu.VMEM((64, 128), jnp.float32),
            pltpu.VMEM((64, 128), jnp.float32),
            pltpu.SemaphoreType.DMA,
            pltpu.SemaphoreType.DMA,
            pltpu.SemaphoreType.DMA,
            pltpu.SemaphoreType.DMA,
            pltpu.SemaphoreType.DMA,
            pltpu.SemaphoreType.DMA,
            pltpu.SemaphoreType.DMA,
            pltpu.SemaphoreType.DMA,
        ],
    )
    xu, xi = detile(user_factors.T, item_factors.T)

    gather = pl.kernel(
        _gather_body,
        out_type=jax.ShapeDtypeStruct((BATCH,), jnp.float32),
        mesh=mesh,
        compiler_params=pltpu.CompilerParams(
            needs_layout_passes=False, use_tc_tiling_on_sc=False),
        scratch_types=[
            pltpu.VMEM((BPW,), jnp.int32),
            pltpu.VMEM((BPW,), jnp.int32),
            pltpu.VMEM((F, BPW), jnp.int32),
            pltpu.VMEM((F, BPW), jnp.int32),
            pltpu.VMEM((F, BPW), jnp.float32),
            pltpu.VMEM((F, BPW), jnp.float32),
            pltpu.VMEM((BPW,), jnp.float32),
            pltpu.VMEM((BPW,), jnp.float32),
            pltpu.VMEM((64 * F,), jnp.float32),
            pltpu.VMEM((64 * F,), jnp.float32),
            pltpu.VMEM((BPW,), jnp.float32),
            pltpu.SemaphoreType.DMA,
            pltpu.SemaphoreType.DMA,
            pltpu.SemaphoreType.DMA,
            pltpu.SemaphoreType.DMA,
        ],
    )
    auxu = user_factors[TAIL:, :].T.reshape(-1)
    auxi = item_factors[TAIL:, :].T.reshape(-1)
    return gather(xu.reshape(-1), xi.reshape(-1),
                  user_biases.reshape(-1), item_biases.reshape(-1),
                  user, item, auxu, auxi)


def kernel(user, item, user_factors, item_factors, user_biases, item_biases):
    return _mf(user, item, user_factors, item_factors, user_biases, item_biases)
